# Initial kernel scaffold; baseline (speedup 1.0000x reference)
import jax
import jax.numpy as jnp
from jax.experimental import pallas as pl


def kernel(x, mixture_weight, cate):
    raise NotImplementedError("write your pallas kernel here")


# speedup vs baseline: 334.5082x; 334.5082x over previous
import jax
import jax.numpy as jnp
from jax.experimental import pallas as pl
from jax.experimental.pallas import tpu as pltpu


def _normalize_body(cate_ref, mw_ref, out_ref):
    c = cate_ref[...]
    cm = jnp.max(c, axis=-1, keepdims=True)
    lse = cm + jnp.log(jnp.sum(jnp.exp(c - cm), axis=-1, keepdims=True))
    w = mw_ref[...]
    wm = jnp.max(w, axis=0, keepdims=True)
    wl = w - (wm + jnp.log(jnp.sum(jnp.exp(w - wm), axis=0, keepdims=True)))
    d = c.shape[1]
    out_ref[...] = (c - lse) + (wl / d)[:, :, None]


def _mixture_body(xt_ref, lp_ref, out_ref):
    d, bb = xt_ref.shape
    m, k = lp_ref.shape
    cp = k // d
    xt = xt_ref[...]
    xr = jnp.broadcast_to(xt[:, None, :], (d, cp, bb))
    ci = jax.lax.broadcasted_iota(jnp.int32, (d, cp, bb), 1)
    oh = jnp.where(xr == ci, jnp.float32(1.0), jnp.float32(0.0))
    ohf = oh.reshape(k, bb)
    acc = jax.lax.dot_general(
        lp_ref[...], ohf, (((1,), (0,)), ((), ())),
        preferred_element_type=jnp.float32)
    mx = jnp.max(acc, axis=0, keepdims=True)
    out_ref[...] = mx + jnp.log(jnp.sum(jnp.exp(acc - mx), axis=0, keepdims=True))


def kernel(x, mixture_weight, cate):
    mm, dd, cc = cate.shape
    bsz = x.shape[0]
    cp = ((cc + 7) // 8) * 8
    bb = 256
    nb = bsz // bb

    cate_p = jnp.pad(cate, ((0, 0), (0, 0), (0, cp - cc)),
                     constant_values=-1e30)
    mw = mixture_weight.reshape(mm, 1).astype(jnp.float32)

    logp = pl.pallas_call(
        _normalize_body,
        grid=(1,),
        in_specs=[
            pl.BlockSpec((mm, dd, cp), lambda i: (0, 0, 0)),
            pl.BlockSpec((mm, 1), lambda i: (0, 0)),
        ],
        out_specs=pl.BlockSpec((mm, dd, cp), lambda i: (0, 0, 0)),
        out_shape=jax.ShapeDtypeStruct((mm, dd, cp), jnp.float32),
    )(cate_p, mw)
    logp_flat = logp.reshape(mm, dd * cp)

    xt = x.astype(jnp.int32).T

    out = pl.pallas_call(
        _mixture_body,
        grid=(nb,),
        in_specs=[
            pl.BlockSpec((dd, bb), lambda i: (0, i)),
            pl.BlockSpec((mm, dd * cp), lambda i: (0, 0)),
        ],
        out_specs=pl.BlockSpec((1, bb), lambda i: (0, i)),
        out_shape=jax.ShapeDtypeStruct((1, bsz), jnp.float32),
        compiler_params=pltpu.CompilerParams(
            dimension_semantics=("parallel",),
            vmem_limit_bytes=56 * 1024 * 1024,
        ),
    )(xt, logp_flat)
    return out.reshape(bsz)


# speedup vs baseline: 620.1592x; 1.8539x over previous
import jax
import jax.numpy as jnp
from jax.experimental import pallas as pl
from jax.experimental.pallas import tpu as pltpu


def _body(cate_ref, mw_ref, xt_ref, out_ref, lp_ref):
    bb, d = xt_ref.shape
    m, k = lp_ref.shape
    cp = k // d

    @pl.when(pl.program_id(0) == 0)
    def _():
        c = cate_ref[...]
        cm = jnp.max(c, axis=-1, keepdims=True)
        lse = cm + jnp.log(jnp.sum(jnp.exp(c - cm), axis=-1, keepdims=True))
        w = mw_ref[...]
        wm = jnp.max(w, axis=0, keepdims=True)
        wl = w - (wm + jnp.log(jnp.sum(jnp.exp(w - wm), axis=0, keepdims=True)))
        lp_ref[...] = ((c - lse) + (wl / d)[:, :, None]).reshape(m, k)

    xt = xt_ref[...].T
    xr = jnp.broadcast_to(xt[:, None, :], (d, cp, bb))
    ci = jax.lax.broadcasted_iota(jnp.int32, (d, cp, bb), 1)
    oh = jnp.where(xr == ci, jnp.float32(1.0), jnp.float32(0.0))
    ohf = oh.reshape(k, bb)
    acc = jax.lax.dot_general(
        lp_ref[...], ohf, (((1,), (0,)), ((), ())),
        preferred_element_type=jnp.float32)
    mx = jnp.max(acc, axis=0, keepdims=True)
    out_ref[...] = mx + jnp.log(jnp.sum(jnp.exp(acc - mx), axis=0, keepdims=True))


def kernel(x, mixture_weight, cate):
    mm, dd, cc = cate.shape
    bsz = x.shape[0]
    cp = ((cc + 7) // 8) * 8
    bb = 4096
    nb = bsz // bb

    cate_p = jnp.pad(cate, ((0, 0), (0, 0), (0, cp - cc)),
                     constant_values=-1e30)
    mw = mixture_weight.reshape(mm, 1).astype(jnp.float32)
    xt = x.astype(jnp.int32)

    out = pl.pallas_call(
        _body,
        grid=(nb,),
        in_specs=[
            pl.BlockSpec((mm, dd, cp), lambda i: (0, 0, 0)),
            pl.BlockSpec((mm, 1), lambda i: (0, 0)),
            pl.BlockSpec((bb, dd), lambda i: (i, 0)),
        ],
        out_specs=pl.BlockSpec((1, bb), lambda i: (0, i)),
        out_shape=jax.ShapeDtypeStruct((1, bsz), jnp.float32),
        scratch_shapes=[pltpu.VMEM((mm, dd * cp), jnp.float32)],
        compiler_params=pltpu.CompilerParams(
            dimension_semantics=("arbitrary",),
            vmem_limit_bytes=56 * 1024 * 1024,
        ),
    )(cate_p, mw, xt)
    return out.reshape(bsz)
